# baseline (speedup 1.0000x reference)
import jax
import jax.numpy as jnp
from jax import lax
from jax.experimental import pallas as pl
from jax.experimental.pallas import tpu as pltpu
from jax.experimental.pallas import tpu_sc as plsc

N_FIELDS = 26
VOCAB = 100000
EMB_DIM = 32
BATCH = 16384

NC = 2
NS = 16
NW = NC * NS
TOTAL = BATCH * N_FIELDS
PER_W = TOTAL // NW
CHUNK = 128
NCH = PER_W // CHUNK
ROW_W = 128
PACK = ROW_W // EMB_DIM
N_ROWS = N_FIELDS * VOCAB // PACK


def _gather_body(idx_hbm, tab_hbm, out_hbm, idx_v, row_v,
                 rows0, rows1, out0, out1, gsem0, gsem1, wsem0, wsem1):
    wid = lax.axis_index("s") * NC + lax.axis_index("c")
    base = wid * PER_W
    iota = lax.iota(jnp.int32, 16)

    pltpu.sync_copy(idx_hbm.at[pl.ds(base, PER_W)], idx_v)

    def row_calc(i, carry):
        v = idx_v[pl.ds(i * 16, 16)]
        row_v[pl.ds(i * 16, 16)] = v >> 2
        return carry

    lax.fori_loop(0, PER_W // 16, row_calc, 0)

    def fire_gather(c, rows_b, sem):
        pltpu.async_copy(tab_hbm.at[row_v.at[pl.ds(c * CHUNK, CHUNK)]],
                         rows_b, sem)

    def wait_gather(c, rows_b, sem):
        pltpu.make_async_copy(tab_hbm.at[row_v.at[pl.ds(c * CHUNK, CHUNK)]],
                              rows_b, sem).wait()

    def fire_wout(c, out_b, sem):
        pltpu.async_copy(out_b, out_hbm.at[pl.ds(base + c * CHUNK, CHUNK)],
                         sem)

    def wait_wout(c, out_b, sem):
        pltpu.make_async_copy(out_b,
                              out_hbm.at[pl.ds(base + c * CHUNK, CHUNK)],
                              sem).wait()

    def extract(c, rows_b, out_b):
        def jb_body(jb, carry):
            idxv = idx_v[pl.ds(c * CHUNK + jb * 16, 16)]
            suboff = (idxv & 3) * EMB_DIM
            rowsel = jb * 16 + iota
            for d in range(EMB_DIM):
                vals = plsc.load_gather(rows_b, [rowsel, suboff + d])
                plsc.store_scatter(out_b,
                                   [rowsel, jnp.full((16,), d, jnp.int32)],
                                   vals)
            return carry

        lax.fori_loop(0, CHUNK // 16, jb_body, 0)

    fire_gather(0, rows0, gsem0)

    def pair(j, carry):
        c0 = 2 * j
        c1 = 2 * j + 1

        fire_gather(c1, rows1, gsem1)
        wait_gather(c0, rows0, gsem0)

        @pl.when(j > 0)
        def _():
            wait_wout(c0 - 2, out0, wsem0)
        extract(c0, rows0, out0)
        fire_wout(c0, out0, wsem0)

        @pl.when(c0 + 2 < NCH)
        def _():
            fire_gather(c0 + 2, rows0, gsem0)
        wait_gather(c1, rows1, gsem1)

        @pl.when(j > 0)
        def _():
            wait_wout(c1 - 2, out1, wsem1)
        extract(c1, rows1, out1)
        fire_wout(c1, out1, wsem1)
        return carry

    lax.fori_loop(0, NCH // 2, pair, 0)
    wait_wout(NCH - 2, out0, wsem0)
    wait_wout(NCH - 1, out1, wsem1)


def kernel(indices, tables):
    tab = tables.reshape(N_ROWS, ROW_W)
    offsets = jnp.arange(N_FIELDS, dtype=jnp.int32) * VOCAB
    flat_idx = (indices.astype(jnp.int32) + offsets[None, :]).reshape(TOTAL)

    mesh = plsc.VectorSubcoreMesh(core_axis_name="c", subcore_axis_name="s")
    out = pl.kernel(
        _gather_body,
        mesh=mesh,
        out_type=jax.ShapeDtypeStruct((TOTAL, EMB_DIM), jnp.float32),
        scratch_types=[
            pltpu.VMEM((PER_W,), jnp.int32),
            pltpu.VMEM((PER_W,), jnp.int32),
            pltpu.VMEM((CHUNK, ROW_W), jnp.float32),
            pltpu.VMEM((CHUNK, ROW_W), jnp.float32),
            pltpu.VMEM((CHUNK, EMB_DIM), jnp.float32),
            pltpu.VMEM((CHUNK, EMB_DIM), jnp.float32),
            pltpu.SemaphoreType.DMA,
            pltpu.SemaphoreType.DMA,
            pltpu.SemaphoreType.DMA,
            pltpu.SemaphoreType.DMA,
        ],
        compiler_params=pltpu.CompilerParams(use_tc_tiling_on_sc=False,
                                             needs_layout_passes=False),
    )(flat_idx, tab)
    return out.reshape(BATCH, N_FIELDS * EMB_DIM)


# speedup vs baseline: 1.0007x; 1.0007x over previous
import jax
import jax.numpy as jnp
from jax import lax
from jax.experimental import pallas as pl
from jax.experimental.pallas import tpu as pltpu
from jax.experimental.pallas import tpu_sc as plsc

N_FIELDS = 26
VOCAB = 100000
EMB_DIM = 32
BATCH = 16384

NC = 2
NS = 16
NW = NC * NS
TOTAL = BATCH * N_FIELDS
PER_W = TOTAL // NW
CHUNK = 128
NCH = PER_W // CHUNK
ROW_W = 128
PACK = ROW_W // EMB_DIM
N_ROWS = N_FIELDS * VOCAB // PACK
OUT_ROWS = TOTAL * EMB_DIM // ROW_W
ORC = CHUNK * EMB_DIM // ROW_W


def _gather_body(idx_hbm, tab_hbm, out_hbm, idx_v, row_v,
                 rows0, rows1, out0, out1, gsem0, gsem1, wsem0, wsem1):
    wid = lax.axis_index("s") * NC + lax.axis_index("c")
    base = wid * PER_W
    obase = wid * (PER_W * EMB_DIM // ROW_W)
    iota = lax.iota(jnp.int32, 16)

    pltpu.sync_copy(idx_hbm.at[pl.ds(base, PER_W)], idx_v)

    def row_calc(i, carry):
        v = idx_v[pl.ds(i * 16, 16)]
        row_v[pl.ds(i * 16, 16)] = v >> 2
        return carry

    lax.fori_loop(0, PER_W // 16, row_calc, 0)

    def fire_gather(c, rows_b, sem):
        pltpu.async_copy(tab_hbm.at[row_v.at[pl.ds(c * CHUNK, CHUNK)]],
                         rows_b, sem)

    def wait_gather(c, rows_b, sem):
        pltpu.make_async_copy(tab_hbm.at[row_v.at[pl.ds(c * CHUNK, CHUNK)]],
                              rows_b, sem).wait()

    def fire_wout(c, out_b, sem):
        pltpu.async_copy(out_b, out_hbm.at[pl.ds(obase + c * ORC, ORC)], sem)

    def wait_wout(c, out_b, sem):
        pltpu.make_async_copy(out_b,
                              out_hbm.at[pl.ds(obase + c * ORC, ORC)],
                              sem).wait()

    def extract(c, rows_b, out_b):
        def jb_body(jb, carry):
            idxv = idx_v[pl.ds(c * CHUNK + jb * 16, 16)]
            suboff = (idxv & 3) * EMB_DIM
            rowsel = jb * 16 + iota
            orow = rowsel >> 2
            ocol = (rowsel & 3) * EMB_DIM
            for d in range(EMB_DIM):
                vals = plsc.load_gather(rows_b, [rowsel, suboff + d])
                plsc.store_scatter(out_b, [orow, ocol + d], vals)
            return carry

        lax.fori_loop(0, CHUNK // 16, jb_body, 0)

    fire_gather(0, rows0, gsem0)

    def pair(j, carry):
        c0 = 2 * j
        c1 = 2 * j + 1

        fire_gather(c1, rows1, gsem1)
        wait_gather(c0, rows0, gsem0)

        @pl.when(j > 0)
        def _():
            wait_wout(c0 - 2, out0, wsem0)
        extract(c0, rows0, out0)
        fire_wout(c0, out0, wsem0)

        @pl.when(c0 + 2 < NCH)
        def _():
            fire_gather(c0 + 2, rows0, gsem0)
        wait_gather(c1, rows1, gsem1)

        @pl.when(j > 0)
        def _():
            wait_wout(c1 - 2, out1, wsem1)
        extract(c1, rows1, out1)
        fire_wout(c1, out1, wsem1)
        return carry

    lax.fori_loop(0, NCH // 2, pair, 0)
    wait_wout(NCH - 2, out0, wsem0)
    wait_wout(NCH - 1, out1, wsem1)


def kernel(indices, tables):
    tab = tables.reshape(N_ROWS, ROW_W)
    offsets = jnp.arange(N_FIELDS, dtype=jnp.int32) * VOCAB
    flat_idx = (indices.astype(jnp.int32) + offsets[None, :]).reshape(TOTAL)

    mesh = plsc.VectorSubcoreMesh(core_axis_name="c", subcore_axis_name="s")
    out = pl.kernel(
        _gather_body,
        mesh=mesh,
        out_type=jax.ShapeDtypeStruct((OUT_ROWS, ROW_W), jnp.float32),
        scratch_types=[
            pltpu.VMEM((PER_W,), jnp.int32),
            pltpu.VMEM((PER_W,), jnp.int32),
            pltpu.VMEM((CHUNK, ROW_W), jnp.float32),
            pltpu.VMEM((CHUNK, ROW_W), jnp.float32),
            pltpu.VMEM((ORC, ROW_W), jnp.float32),
            pltpu.VMEM((ORC, ROW_W), jnp.float32),
            pltpu.SemaphoreType.DMA,
            pltpu.SemaphoreType.DMA,
            pltpu.SemaphoreType.DMA,
            pltpu.SemaphoreType.DMA,
        ],
        compiler_params=pltpu.CompilerParams(use_tc_tiling_on_sc=True,
                                             needs_layout_passes=False),
    )(flat_idx, tab)
    return out.reshape(BATCH, N_FIELDS * EMB_DIM)


# speedup vs baseline: 1.0306x; 1.0300x over previous
import jax
import jax.numpy as jnp
from jax import lax
from jax.experimental import pallas as pl
from jax.experimental.pallas import tpu as pltpu
from jax.experimental.pallas import tpu_sc as plsc

N_FIELDS = 26
VOCAB = 100000
EMB_DIM = 32
BATCH = 16384

NC = 2
NS = 16
NW = NC * NS
TOTAL = BATCH * N_FIELDS
PER_W = TOTAL // NW

ROW_W = 128
FLAT_ROWS = N_FIELDS * VOCAB * EMB_DIM // ROW_W
VB = 128
NTC = VOCAB // VB
TAIL = VOCAB - NTC * VB
NBLK = N_FIELDS * NTC
NIT = 636
ORPB = VB * EMB_DIM // ROW_W
FPB = VOCAB // 4

CHUNK = 256
NCH = PER_W // CHUNK
K = 2
NGRP = NCH // K


def _transpose_body(tab_hbm, tails_hbm, out_hbm, in0, in1, st0, st1,
                    isem0, isem1, osem0, osem1):
    wid = lax.axis_index("s") * NC + lax.axis_index("c")
    iota = lax.iota(jnp.int32, 16)
    orow = [vb * 4 + (iota >> 2) for vb in range(8)]
    ocol = (iota & 3) * EMB_DIM

    def bid_of(i):
        return wid + NW * i

    def active(i):
        return bid_of(i) < NBLK

    def fire_in(i, in_b, sem):
        b = bid_of(i)
        f = b // NTC
        tc = b - f * NTC
        pltpu.async_copy(tab_hbm.at[f, :, pl.ds(tc * VB, VB)], in_b, sem)

    def wait_in(in_b, sem):
        pltpu.make_async_copy(tab_hbm.at[0, :, pl.ds(0, VB)], in_b, sem).wait()

    def fire_out(i, st_b, sem):
        b = bid_of(i)
        f = b // NTC
        tc = b - f * NTC
        pltpu.async_copy(st_b, out_hbm.at[pl.ds(f * FPB + tc * ORPB, ORPB)],
                         sem)

    def wait_out(st_b, sem):
        pltpu.make_async_copy(st_b, out_hbm.at[pl.ds(0, ORPB)], sem).wait()

    def transpose_block(in_b, st_b):
        for d in range(EMB_DIM):
            oc = ocol + d
            vals = [in_b[d, pl.ds(vb * 16, 16)] for vb in range(8)]
            for vb in range(8):
                plsc.store_scatter(st_b, [orow[vb], oc], vals[vb])

    @pl.when(active(0))
    def _():
        fire_in(0, in0, isem0)

    def pair(j, carry):
        i0 = 2 * j
        i1 = 2 * j + 1

        @pl.when(active(i1))
        def _():
            fire_in(i1, in1, isem1)

        @pl.when(active(i0))
        def _():
            wait_in(in0, isem0)

            @pl.when(j > 0)
            def _():
                wait_out(st0, osem0)
            transpose_block(in0, st0)
            fire_out(i0, st0, osem0)

        @pl.when(active(i0 + 2))
        def _():
            fire_in(i0 + 2, in0, isem0)

        @pl.when(active(i1))
        def _():
            wait_in(in1, isem1)

            @pl.when(j > 0)
            def _():
                wait_out(st1, osem1)
            transpose_block(in1, st1)
            fire_out(i1, st1, osem1)
        return carry

    lax.fori_loop(0, NIT // 2, pair, 0)
    wait_out(st0, osem0)
    wait_out(st1, osem1)

    @pl.when(wid < N_FIELDS)
    def _():
        pltpu.sync_copy(tails_hbm.at[wid], st0.at[pl.ds(0, 8)])
        pltpu.sync_copy(st0.at[pl.ds(0, 8)],
                        out_hbm.at[pl.ds(wid * FPB + NTC * ORPB, 8)])


def _gather_body(idx_hbm, tab_hbm, out_hbm, idx_v, rows_v,
                 gsem0, gsem1, wsem0, wsem1):
    wid = lax.axis_index("s") * NC + lax.axis_index("c")
    base_chunk = wid * NCH

    def fire_gathers(grp, set_, sem):
        for b in range(K):
            pltpu.async_copy(tab_hbm.at[idx_v.at[grp * K + b]],
                             rows_v.at[set_, b], sem)

    def wait_gathers(set_, sem):
        for b in range(K):
            pltpu.make_async_copy(tab_hbm.at[idx_v.at[0]],
                                  rows_v.at[set_, b], sem).wait()

    def fire_wouts(grp, set_, sem):
        for b in range(K):
            c = base_chunk + grp * K + b
            pltpu.async_copy(rows_v.at[set_, b],
                             out_hbm.at[pl.ds(c * CHUNK, CHUNK)], sem)

    def wait_wouts(set_, sem):
        for b in range(K):
            pltpu.make_async_copy(rows_v.at[set_, b],
                                  out_hbm.at[pl.ds(0, CHUNK)], sem).wait()

    pltpu.sync_copy(idx_hbm.at[pl.ds(wid * NCH, NCH)], idx_v)
    fire_gathers(0, 0, gsem0)

    def pair(j, carry):
        g0 = 2 * j
        g1 = 2 * j + 1

        @pl.when(j > 0)
        def _():
            wait_wouts(1, wsem1)
        fire_gathers(g1, 1, gsem1)
        wait_gathers(0, gsem0)
        fire_wouts(g0, 0, wsem0)
        wait_wouts(0, wsem0)

        @pl.when(g0 + 2 < NGRP)
        def _():
            fire_gathers(g0 + 2, 0, gsem0)
        wait_gathers(1, gsem1)
        fire_wouts(g1, 1, wsem1)
        return carry

    lax.fori_loop(0, NGRP // 2, pair, 0)
    wait_wouts(1, wsem1)


def kernel(indices, tables):
    tab_native = jnp.transpose(tables, (0, 2, 1))
    tails = tables[:, NTC * VB:, :].reshape(N_FIELDS, 8, ROW_W)
    mesh = plsc.VectorSubcoreMesh(core_axis_name="c", subcore_axis_name="s")

    t128 = pl.kernel(
        _transpose_body,
        mesh=mesh,
        out_type=jax.ShapeDtypeStruct((FLAT_ROWS, ROW_W), jnp.float32),
        scratch_types=[
            pltpu.VMEM((EMB_DIM, VB), jnp.float32),
            pltpu.VMEM((EMB_DIM, VB), jnp.float32),
            pltpu.VMEM((ORPB, ROW_W), jnp.float32),
            pltpu.VMEM((ORPB, ROW_W), jnp.float32),
            pltpu.SemaphoreType.DMA,
            pltpu.SemaphoreType.DMA,
            pltpu.SemaphoreType.DMA,
            pltpu.SemaphoreType.DMA,
        ],
        compiler_params=pltpu.CompilerParams(use_tc_tiling_on_sc=True,
                                             needs_layout_passes=False),
    )(tab_native, tails)

    flat_tables = t128.reshape(N_FIELDS * VOCAB, EMB_DIM)
    offsets = jnp.arange(N_FIELDS, dtype=jnp.int32) * VOCAB
    flat_idx = (indices.astype(jnp.int32) + offsets[None, :]).reshape(
        TOTAL // CHUNK, CHUNK)

    out = pl.kernel(
        _gather_body,
        mesh=mesh,
        out_type=jax.ShapeDtypeStruct((TOTAL, EMB_DIM), jnp.float32),
        scratch_types=[
            pltpu.VMEM((NCH, CHUNK), jnp.int32),
            pltpu.VMEM((2, K, CHUNK, EMB_DIM), jnp.float32),
            pltpu.SemaphoreType.DMA,
            pltpu.SemaphoreType.DMA,
            pltpu.SemaphoreType.DMA,
            pltpu.SemaphoreType.DMA,
        ],
        compiler_params=pltpu.CompilerParams(use_tc_tiling_on_sc=False),
    )(flat_idx, flat_tables)
    return out.reshape(BATCH, N_FIELDS * EMB_DIM)
